# baseline (speedup 1.0000x reference)
import functools

import jax
import jax.numpy as jnp
from jax import lax
from jax.experimental import pallas as pl
from jax.experimental.pallas import tpu as pltpu
from jax.experimental.pallas import tpu_sc as plsc

_PRUNE_RATIO = 0.5

_NC = 2
_NS = 16
_NW = _NC * _NS
_L = 16

_NB = 1024
_UNROLL = 4

_SROWS = 125
_RC = 256


def _flat(ref, i):
    return ref[i >> 2, pl.ds((i & 3) * _L, _L)]


def _flat_store(ref, i, v):
    ref[i >> 2, pl.ds((i & 3) * _L, _L)] = v


def _body(n_rows, n_idx, j0, sbits_hbm, wbits_hbm, idx_hbm, out_hbm,
          buf0, buf1, s0, s1, s2, s3, hist_1, idxscr, mbuf, spart,
          dma0, dma1, dw0, dw1, ds0, ds1, do0, do1):
    c = lax.axis_index("c")
    s = lax.axis_index("s")
    wid = s * _NC + c
    stripes = (s0, s1, s2, s3)
    lane = lax.iota(jnp.int32, _L)
    ones = jnp.ones((_L,), jnp.int32)
    zeros = jnp.zeros((_L,), jnp.int32)

    pltpu.sync_copy(idx_hbm.at[pl.ds(wid * (n_idx // _NW), n_idx // _NW)],
                    idxscr)

    rows_per_tile = n_rows // _NS
    nchunks = rows_per_tile // _SROWS
    row_base = s * rows_per_tile
    scan_sems = (dma0, dma1)
    scan_bufs = (buf0, buf1)

    def scan_start(chunk_i, b):
        return pltpu.async_copy(
            sbits_hbm.at[pl.ds(row_base + chunk_i * _SROWS, _SROWS)],
            scan_bufs[b], scan_sems[b])

    def scan_wait(b):
        pltpu.make_async_copy(
            sbits_hbm.at[pl.ds(row_base, _SROWS)],
            scan_bufs[b], scan_sems[b]).wait()

    lo = jnp.zeros((_L,), jnp.int32)
    j_rem = jnp.int32(j0)

    for pass_i, shift in enumerate((20, 10, 0)):
        masked = pass_i != 0
        hi = lo + (_NB << shift)

        @plsc.parallel_loop(0, _NB)
        def _(i):
            for u in range(_UNROLL):
                _flat_store(stripes[u], i, zeros)

        scan_start(0, 0)

        def process(buf):
            @plsc.parallel_loop(0, _SROWS * 64 // _L // _UNROLL, 1, unroll=2)
            def _(i):
                for u in range(_UNROLL):
                    x = _flat(buf, i * _UNROLL + u)
                    bits = x & jnp.int32(0x7FFFFFFF)
                    b = (bits - lo) >> shift if shift else (bits - lo)
                    b = jnp.clip(b, 0, _NB - 1)
                    addr = b * _L + lane
                    row = addr >> 6
                    col = addr & 63
                    if masked:
                        in_rng = (bits >= lo) & (bits < hi)
                        plsc.addupdate_scatter(
                            stripes[u], [row, col], ones, mask=in_rng)
                    else:
                        plsc.addupdate_scatter(stripes[u], [row, col], ones)

        def pair_body(k, carry):
            scan_start(2 * k + 1, 1)
            scan_wait(0)
            process(buf0)
            nxt = jnp.minimum(2 * k + 2, nchunks - 1)
            scan_start(nxt, 0)
            scan_wait(1)
            process(buf1)
            return carry

        lax.fori_loop(0, nchunks // 2, pair_body, 0)
        scan_wait(0)

        @plsc.parallel_loop(0, _NB // _L)
        def _(g):
            bvec = lax.iota(jnp.int32, _L) + g * _L
            vals = []
            for u in range(_UNROLL):
                for l in range(_L):
                    flat = bvec * _L + l
                    vals.append(plsc.load_gather(
                        stripes[u], [flat >> 6, flat & 63]))
            while len(vals) > 1:
                vals = ([vals[i] + vals[i + 1]
                         for i in range(0, len(vals) - 1, 2)]
                        + ([vals[-1]] if len(vals) % 2 else []))
            hist_1[pl.ds(g * _L, _L)] = vals[0]

        pltpu.sync_copy(hist_1, spart.at[s])
        plsc.subcore_barrier()
        pltpu.sync_copy(spart, mbuf)
        plsc.subcore_barrier()

        @plsc.parallel_loop(0, _NB // _L)
        def _(g):
            vals = [mbuf[r, pl.ds(g * _L, _L)] for r in range(_NS)]
            while len(vals) > 1:
                vals = [vals[i] + vals[i + 1] for i in range(0, len(vals), 2)]
            hist_1[pl.ds(g * _L, _L)] = vals[0]

        def sel_body(g, carry):
            below, bsel, found = carry
            v = hist_1[pl.ds(g * _L, _L)]
            cum = below + plsc.cumsum(v)
            take = (cum <= j_rem) & jnp.logical_not(found)
            below = below + jnp.sum(jnp.where(take, v, 0))
            ntaken = jnp.sum(take.astype(jnp.int32))
            bsel = bsel + ntaken
            found = jnp.logical_or(found, ntaken < _L)
            return below, bsel, found

        below, bsel, _f = lax.fori_loop(
            0, _NB // _L, sel_body, (jnp.int32(0), jnp.int32(0), False))
        bsel = jnp.minimum(bsel, _NB - 1)
        j_rem = j_rem - below
        lo = lo + (bsel << shift)

    tvec = lo

    rows_per_w = n_idx // _NW
    g_chunks = rows_per_w // _RC
    g_base = wid * rows_per_w
    wscr = (s0, s1)
    sscr = (s2, s3)
    wsem = (dw0, dw1)
    ssem = (ds0, ds1)
    osem = (do0, do1)

    def g_start(ci):
        b = ci % 2
        sl = idxscr.at[pl.ds(ci * _RC, _RC)]
        pltpu.async_copy(wbits_hbm.at[sl], wscr[b], wsem[b])
        pltpu.async_copy(sbits_hbm.at[sl], sscr[b], ssem[b])

    def g_wait(ci):
        b = ci % 2
        sl = idxscr.at[pl.ds(ci * _RC, _RC)]
        pltpu.make_async_copy(wbits_hbm.at[sl], wscr[b], wsem[b]).wait()
        pltpu.make_async_copy(sbits_hbm.at[sl], sscr[b], ssem[b]).wait()

    def out_start(ci):
        b = ci % 2
        pltpu.async_copy(wscr[b],
                         out_hbm.at[pl.ds(g_base + ci * _RC, _RC)], osem[b])

    def out_wait(ci):
        b = ci % 2
        pltpu.make_async_copy(
            wscr[b], out_hbm.at[pl.ds(g_base + ci * _RC, _RC)],
            osem[b]).wait()

    g_start(0)
    for ci in range(g_chunks):
        b = ci % 2
        if ci + 1 < g_chunks:
            if ci >= 1:
                out_wait(ci - 1)
            g_start(ci + 1)
        g_wait(ci)
        wb, sb = wscr[b], sscr[b]

        @plsc.parallel_loop(0, _RC * 64 // _L, 1, unroll=2)
        def _(i):
            svb = _flat(sb, i)
            abits = svb & jnp.int32(0x7FFFFFFF)
            keep = abits >= tvec
            a = plsc.bitcast(abits, jnp.float32)
            binary = jnp.where(keep, jnp.float32(1.0), jnp.float32(0.0))
            m = a + (binary - a)
            w = plsc.bitcast(_flat(wb, i), jnp.float32)
            _flat_store(wb, i, plsc.bitcast(w * m, jnp.int32))

        out_start(ci)
    out_wait(g_chunks - 2)
    out_wait(g_chunks - 1)


def kernel(input, weight, scores):
    n_rows, dim = weight.shape
    n_total = n_rows * dim
    j0 = int((1.0 - _PRUNE_RATIO) * n_total)
    idx = input.reshape(-1)
    n_idx = idx.shape[0]

    sbits = lax.bitcast_convert_type(scores, jnp.int32)
    wbits = lax.bitcast_convert_type(weight, jnp.int32)

    mesh = plsc.VectorSubcoreMesh(core_axis_name="c", subcore_axis_name="s")
    out_i32 = pl.kernel(
        functools.partial(_body, n_rows, n_idx, j0),
        out_type=jax.ShapeDtypeStruct((n_idx, dim), jnp.int32),
        mesh=mesh,
        compiler_params=pltpu.CompilerParams(needs_layout_passes=False,
                                             use_tc_tiling_on_sc=False),
        scratch_types=[
            pltpu.VMEM((_SROWS, 64), jnp.int32),
            pltpu.VMEM((_SROWS, 64), jnp.int32),
            pltpu.VMEM((_RC, 64), jnp.int32),
            pltpu.VMEM((_RC, 64), jnp.int32),
            pltpu.VMEM((_RC, 64), jnp.int32),
            pltpu.VMEM((_RC, 64), jnp.int32),
            pltpu.VMEM((_NB,), jnp.int32),
            pltpu.VMEM((n_idx // _NW,), jnp.int32),
            pltpu.VMEM((_NS, _NB), jnp.int32),
            pltpu.VMEM_SHARED((_NS, _NB), jnp.int32),
        ] + [pltpu.SemaphoreType.DMA] * 8,
    )(sbits, wbits, idx)

    out = lax.bitcast_convert_type(out_i32, jnp.float32)
    return out.reshape(input.shape + (dim,))


# speedup vs baseline: 1.0573x; 1.0573x over previous
import functools

import jax
import jax.numpy as jnp
from jax import lax
from jax.experimental import pallas as pl
from jax.experimental.pallas import tpu as pltpu
from jax.experimental.pallas import tpu_sc as plsc

_PRUNE_RATIO = 0.5

_NC = 2
_NS = 16
_NW = _NC * _NS
_L = 16

_NBUCKETS = 1024


_UNROLL = 4


def _hist_body(shift, masked, n_per_tile, chunk, scores_hbm, lo_hbm, out_hbm,
               buf0, buf1, lo_v, hist_l, hist_1, sem0, sem1):
    c = lax.axis_index("c")
    s = lax.axis_index("s")
    wid = s * _NC + c

    pltpu.sync_copy(lo_hbm, lo_v)
    lo = lo_v[...]
    hi = lo + (_NBUCKETS << shift)
    lane = lax.iota(jnp.int32, _L)
    ones = jnp.ones((_L,), jnp.int32)
    zeros = jnp.zeros((_L,), jnp.int32)

    @plsc.parallel_loop(0, _NBUCKETS * _UNROLL)
    def _(i):
        hist_l[pl.ds(i * _L, _L)] = zeros

    base = wid * n_per_tile
    nchunks = n_per_tile // chunk
    bufs = (buf0, buf1)
    sems = (sem0, sem1)

    def start(ci):
        return pltpu.async_copy(
            scores_hbm.at[pl.ds(base + ci * chunk, chunk)],
            bufs[ci % 2], sems[ci % 2])

    handles = {0: start(0)}
    for ci in range(nchunks):
        handles.pop(ci).wait()
        if ci + 1 < nchunks:
            handles[ci + 1] = start(ci + 1)
        bufc = bufs[ci % 2]

        @plsc.parallel_loop(0, chunk // _L, 1, unroll=_UNROLL)
        def _(i):
            x = bufc[pl.ds(i * _L, _L)]
            bits = x & jnp.int32(0x7FFFFFFF)
            b = (bits - lo) >> shift if shift else (bits - lo)
            b = jnp.clip(b, 0, _NBUCKETS - 1)
            stripe = (i & (_UNROLL - 1)) * (_NBUCKETS * _L)
            addr = b * _L + lane + stripe
            if masked:
                in_rng = (bits >= lo) & (bits < hi)
                plsc.addupdate_scatter(hist_l, [addr], ones, mask=in_rng)
            else:
                plsc.addupdate_scatter(hist_l, [addr], ones)

    @plsc.parallel_loop(0, _NBUCKETS // _L)
    def _(g):
        bvec = lax.iota(jnp.int32, _L) + g * _L
        vals = [plsc.load_gather(hist_l, [(u * _NBUCKETS + bvec) * _L + l])
                for u in range(_UNROLL) for l in range(_L)]
        while len(vals) > 1:
            vals = ([vals[i] + vals[i + 1] for i in range(0, len(vals) - 1, 2)]
                    + ([vals[-1]] if len(vals) % 2 else []))
        hist_1[pl.ds(g * _L, _L)] = vals[0]

    pltpu.sync_copy(hist_1, out_hbm.at[wid])


def _hist_pass(shift, masked, n_total, chunk):
    n_per_tile = n_total // _NW
    mesh = plsc.VectorSubcoreMesh(core_axis_name="c", subcore_axis_name="s")
    return pl.kernel(
        functools.partial(_hist_body, shift, masked, n_per_tile, chunk),
        out_type=jax.ShapeDtypeStruct((_NW, _NBUCKETS), jnp.int32),
        mesh=mesh,
        compiler_params=pltpu.CompilerParams(needs_layout_passes=False),
        scratch_types=[
            pltpu.VMEM((chunk,), jnp.int32),
            pltpu.VMEM((chunk,), jnp.int32),
            pltpu.VMEM((_L,), jnp.int32),
            pltpu.VMEM((_NBUCKETS * _L * _UNROLL,), jnp.int32),
            pltpu.VMEM((_NBUCKETS,), jnp.int32),
            pltpu.SemaphoreType.DMA,
            pltpu.SemaphoreType.DMA,
        ],
    )


_RC = 256


def _flat(ref, i):
    return ref[i >> 2, pl.ds((i & 3) * _L, _L)]


def _mgather_body(b_per_w, wbits_hbm, sbits_hbm, idx_hbm, tv_hbm, out_hbm,
                  w0, w1, q0, q1, idxscr, tv_v, dw0, dw1, ds0, ds1, do0, do1):
    c = lax.axis_index("c")
    s = lax.axis_index("s")
    wid = s * _NC + c
    base = wid * b_per_w
    g_chunks = b_per_w // _RC

    pltpu.sync_copy(tv_hbm, tv_v)
    tvec = tv_v[...]
    pltpu.sync_copy(idx_hbm.at[pl.ds(base, b_per_w)], idxscr)

    wscr = (w0, w1)
    sscr = (q0, q1)
    wsem = (dw0, dw1)
    ssem = (ds0, ds1)
    osem = (do0, do1)

    def g_start(ci):
        b = ci % 2
        sl = idxscr.at[pl.ds(ci * _RC, _RC)]
        pltpu.async_copy(wbits_hbm.at[sl], wscr[b], wsem[b])
        pltpu.async_copy(sbits_hbm.at[sl], sscr[b], ssem[b])

    def g_wait(ci):
        b = ci % 2
        sl = idxscr.at[pl.ds(ci * _RC, _RC)]
        pltpu.make_async_copy(wbits_hbm.at[sl], wscr[b], wsem[b]).wait()
        pltpu.make_async_copy(sbits_hbm.at[sl], sscr[b], ssem[b]).wait()

    def out_start(ci):
        b = ci % 2
        pltpu.async_copy(wscr[b],
                         out_hbm.at[pl.ds(base + ci * _RC, _RC)], osem[b])

    def out_wait(ci):
        b = ci % 2
        pltpu.make_async_copy(
            wscr[b], out_hbm.at[pl.ds(base + ci * _RC, _RC)], osem[b]).wait()

    g_start(0)
    for ci in range(g_chunks):
        b = ci % 2
        if ci + 1 < g_chunks:
            if ci >= 1:
                out_wait(ci - 1)
            g_start(ci + 1)
        g_wait(ci)
        wb, sb = wscr[b], sscr[b]

        @plsc.parallel_loop(0, _RC * 64 // _L, 1, unroll=2)
        def _(i):
            svb = _flat(sb, i)
            abits = svb & jnp.int32(0x7FFFFFFF)
            keep = abits >= tvec
            a = plsc.bitcast(abits, jnp.float32)
            binary = jnp.where(keep, jnp.float32(1.0), jnp.float32(0.0))
            m = a + (binary - a)
            w = plsc.bitcast(_flat(wb, i), jnp.float32)
            wb[i >> 2, pl.ds((i & 3) * _L, _L)] = plsc.bitcast(w * m,
                                                               jnp.int32)

        out_start(ci)
    out_wait(g_chunks - 2)
    out_wait(g_chunks - 1)


def _masked_gather(wbits, sbits, idx_flat, tvec16):
    nrows, dim = wbits.shape
    n_idx = idx_flat.shape[0]
    b_per_w = n_idx // _NW
    mesh = plsc.VectorSubcoreMesh(core_axis_name="c", subcore_axis_name="s")
    return pl.kernel(
        functools.partial(_mgather_body, b_per_w),
        out_type=jax.ShapeDtypeStruct((n_idx, dim), jnp.int32),
        mesh=mesh,
        compiler_params=pltpu.CompilerParams(needs_layout_passes=False,
                                             use_tc_tiling_on_sc=False),
        scratch_types=[
            pltpu.VMEM((_RC, 64), jnp.int32),
            pltpu.VMEM((_RC, 64), jnp.int32),
            pltpu.VMEM((_RC, 64), jnp.int32),
            pltpu.VMEM((_RC, 64), jnp.int32),
            pltpu.VMEM((b_per_w,), jnp.int32),
            pltpu.VMEM((_L,), jnp.int32),
        ] + [pltpu.SemaphoreType.DMA] * 6,
    )(wbits, sbits, idx_flat, tvec16)


def kernel(input, weight, scores):
    n_total = weight.shape[0] * weight.shape[1]
    j = int((1.0 - _PRUNE_RATIO) * n_total)

    sbits = lax.bitcast_convert_type(scores.reshape(-1), jnp.int32)
    lo = jnp.zeros((_L,), jnp.int32)
    j_rem = jnp.int32(j)
    for shift in (20, 10, 0):
        part = _hist_pass(shift, shift != 20, n_total, 20000)(sbits, lo)
        h = jnp.sum(part, axis=0)
        cum = jnp.cumsum(h)
        b = jnp.minimum(jnp.sum((cum <= j_rem).astype(jnp.int32)),
                        jnp.int32(_NBUCKETS - 1))
        below = jnp.take(cum, b) - jnp.take(h, b)
        j_rem = j_rem - below
        lo = lo + (b << shift).astype(jnp.int32)

    t_bits = lo[0]
    sbits2d = lax.bitcast_convert_type(scores, jnp.int32)
    wbits2d = lax.bitcast_convert_type(weight, jnp.int32)
    tvec16 = jnp.full((_L,), t_bits, jnp.int32)
    out_i32 = _masked_gather(wbits2d, sbits2d, input.reshape(-1), tvec16)
    out = lax.bitcast_convert_type(out_i32, jnp.float32)
    return out.reshape(input.shape + (weight.shape[1],))


# speedup vs baseline: 1.3619x; 1.2881x over previous
import functools

import jax
import jax.numpy as jnp
from jax import lax
from jax.experimental import pallas as pl
from jax.experimental.pallas import tpu as pltpu
from jax.experimental.pallas import tpu_sc as plsc

_PRUNE_RATIO = 0.5

_NC = 2
_NS = 16
_NW = _NC * _NS
_L = 16

_NBUCKETS = 1024


_UNROLL = 4


def _hist_body(shift, masked, n_per_tile, chunk, scores_hbm, lo_hbm, out_hbm,
               buf0, buf1, lo_v, hist_l, hist_1, sem0, sem1):
    c = lax.axis_index("c")
    s = lax.axis_index("s")
    wid = s * _NC + c

    pltpu.sync_copy(lo_hbm, lo_v)
    lo = lo_v[...]
    hi = lo + (_NBUCKETS << shift)
    lane = lax.iota(jnp.int32, _L)
    ones = jnp.ones((_L,), jnp.int32)
    zeros = jnp.zeros((_L,), jnp.int32)

    @plsc.parallel_loop(0, _NBUCKETS * _UNROLL)
    def _(i):
        hist_l[pl.ds(i * _L, _L)] = zeros

    base = wid * n_per_tile
    nchunks = n_per_tile // chunk
    bufs = (buf0, buf1)
    sems = (sem0, sem1)

    def start(ci):
        return pltpu.async_copy(
            scores_hbm.at[pl.ds(base + ci * chunk, chunk)],
            bufs[ci % 2], sems[ci % 2])

    handles = {0: start(0)}
    for ci in range(nchunks):
        handles.pop(ci).wait()
        if ci + 1 < nchunks:
            handles[ci + 1] = start(ci + 1)
        bufc = bufs[ci % 2]

        @plsc.parallel_loop(0, chunk // _L, 1, unroll=_UNROLL)
        def _(i):
            x = bufc[pl.ds(i * _L, _L)]
            bits = x & jnp.int32(0x7FFFFFFF)
            b = (bits - lo) >> shift if shift else (bits - lo)
            b = jnp.clip(b, 0, _NBUCKETS - 1)
            stripe = (i & (_UNROLL - 1)) * (_NBUCKETS * _L)
            addr = b * _L + lane + stripe
            if masked:
                in_rng = (bits >= lo) & (bits < hi)
                plsc.addupdate_scatter(hist_l, [addr], ones, mask=in_rng)
            else:
                plsc.addupdate_scatter(hist_l, [addr], ones)

    @plsc.parallel_loop(0, _NBUCKETS // _L)
    def _(g):
        bvec = lax.iota(jnp.int32, _L) + g * _L
        vals = [plsc.load_gather(hist_l, [(u * _NBUCKETS + bvec) * _L + l])
                for u in range(_UNROLL) for l in range(_L)]
        while len(vals) > 1:
            vals = ([vals[i] + vals[i + 1] for i in range(0, len(vals) - 1, 2)]
                    + ([vals[-1]] if len(vals) % 2 else []))
        hist_1[pl.ds(g * _L, _L)] = vals[0]

    pltpu.sync_copy(hist_1, out_hbm.at[wid])


def _hist_pass(shift, masked, n_total, chunk):
    n_per_tile = n_total // _NW
    mesh = plsc.VectorSubcoreMesh(core_axis_name="c", subcore_axis_name="s")
    return pl.kernel(
        functools.partial(_hist_body, shift, masked, n_per_tile, chunk),
        out_type=jax.ShapeDtypeStruct((_NW, _NBUCKETS), jnp.int32),
        mesh=mesh,
        compiler_params=pltpu.CompilerParams(needs_layout_passes=False),
        scratch_types=[
            pltpu.VMEM((chunk,), jnp.int32),
            pltpu.VMEM((chunk,), jnp.int32),
            pltpu.VMEM((_L,), jnp.int32),
            pltpu.VMEM((_NBUCKETS * _L * _UNROLL,), jnp.int32),
            pltpu.VMEM((_NBUCKETS,), jnp.int32),
            pltpu.SemaphoreType.DMA,
            pltpu.SemaphoreType.DMA,
        ],
    )


def _mask_body(t_ref, w_ref, s_ref, o_ref):
    t = t_ref[0]
    sv = s_ref[...]
    a = jnp.abs(sv)
    bits = lax.bitcast_convert_type(sv, jnp.int32) & jnp.int32(0x7FFFFFFF)
    binary = jnp.where(bits >= t, jnp.float32(1.0), jnp.float32(0.0))
    m = a + (binary - a)
    o_ref[...] = w_ref[...] * m


def _apply_mask(w_flat, s_flat, t_bits):
    n = w_flat.shape[0]
    block = 128000
    grid = n // block
    return pl.pallas_call(
        _mask_body,
        grid=(grid,),
        in_specs=[
            pl.BlockSpec(memory_space=pltpu.SMEM),
            pl.BlockSpec((block,), lambda i: (i,)),
            pl.BlockSpec((block,), lambda i: (i,)),
        ],
        out_specs=pl.BlockSpec((block,), lambda i: (i,)),
        out_shape=jax.ShapeDtypeStruct((n,), jnp.float32),
    )(jnp.full((1,), t_bits, jnp.int32), w_flat, s_flat)


def _gather_body(b_per_w, gchunk, dim, tbl_hbm, idx_hbm, out_hbm,
                 idx_c, rows_v, sem):
    c = lax.axis_index("c")
    s = lax.axis_index("s")
    wid = s * _NC + c
    base = wid * b_per_w

    def chunk_body(ci, carry):
        row0 = base + ci * gchunk
        pltpu.sync_copy(idx_hbm.at[pl.ds(row0, gchunk)], idx_c)
        pltpu.async_copy(tbl_hbm.at[idx_c], rows_v, sem).wait()
        pltpu.sync_copy(rows_v, out_hbm.at[pl.ds(row0, gchunk)])
        return carry

    lax.fori_loop(0, b_per_w // gchunk, chunk_body, 0)


def _gather(table, idx_flat):
    nrows, dim = table.shape
    n_idx = idx_flat.shape[0]
    b_per_w = n_idx // _NW
    gchunk = 800
    mesh = plsc.VectorSubcoreMesh(core_axis_name="c", subcore_axis_name="s")
    return pl.kernel(
        functools.partial(_gather_body, b_per_w, gchunk, dim),
        out_type=jax.ShapeDtypeStruct((n_idx, dim), jnp.float32),
        mesh=mesh,
        compiler_params=pltpu.CompilerParams(needs_layout_passes=False,
                                             use_tc_tiling_on_sc=False),
        scratch_types=[
            pltpu.VMEM((gchunk,), jnp.int32),
            pltpu.VMEM((gchunk, dim), jnp.float32),
            pltpu.SemaphoreType.DMA,
        ],
    )(table, idx_flat)


def kernel(input, weight, scores):
    n_total = weight.shape[0] * weight.shape[1]
    j = int((1.0 - _PRUNE_RATIO) * n_total)

    sbits = lax.bitcast_convert_type(scores.reshape(-1), jnp.int32)
    lo = jnp.zeros((_L,), jnp.int32)
    j_rem = jnp.int32(j)
    for shift in (20, 10, 0):
        part = _hist_pass(shift, shift != 20, n_total, 20000)(sbits, lo)
        h = jnp.sum(part, axis=0)
        cum = jnp.cumsum(h)
        b = jnp.minimum(jnp.sum((cum <= j_rem).astype(jnp.int32)),
                        jnp.int32(_NBUCKETS - 1))
        below = jnp.take(cum, b) - jnp.take(h, b)
        j_rem = j_rem - below
        lo = lo + (b << shift).astype(jnp.int32)

    t_bits = lo[0]
    masked = _apply_mask(weight.reshape(-1), scores.reshape(-1), t_bits)
    out = _gather(masked.reshape(weight.shape), input.reshape(-1))
    return out.reshape(input.shape + (weight.shape[1],))
